# baseline (speedup 1.0000x reference)
import functools

import jax
import jax.numpy as jnp
from jax import lax
from jax.experimental import pallas as pl
from jax.experimental.pallas import tpu as pltpu
from jax.experimental.pallas import tpu_sc as plsc

_NUM_CORES = 2
_NUM_SUBCORES = 16


@functools.lru_cache(maxsize=None)
def _sc_broadcast(table_words: int, tile_words: int, bs: int):
    nw = _NUM_CORES * _NUM_SUBCORES
    assert bs % nw == 0, (bs, nw)
    assert tile_words % 8 == 0, tile_words
    b_per_w = bs // nw
    mesh = plsc.VectorSubcoreMesh(core_axis_name="c", subcore_axis_name="s")

    rep = 8
    assert b_per_w % rep == 0, (b_per_w, rep)

    @functools.partial(
        pl.kernel,
        out_type=jax.ShapeDtypeStruct((bs * tile_words,), jnp.float32),
        mesh=mesh,
        scratch_types=[
            pltpu.VMEM((rep * tile_words,), jnp.float32),
            pltpu.SemaphoreType.DMA,
        ],
    )
    def k(table_hbm, out_hbm, tile_v, sem):
        wid = lax.axis_index("s") * _NUM_CORES + lax.axis_index("c")
        fills = [
            pltpu.async_copy(
                table_hbm.at[pl.ds(0, tile_words)],
                tile_v.at[pl.ds(r * tile_words, tile_words)],
                sem,
            )
            for r in range(rep)
        ]
        for h in fills:
            h.wait()
        base = wid * b_per_w * tile_words
        blk = rep * tile_words
        handles = [
            pltpu.async_copy(tile_v, out_hbm.at[pl.ds(base + j * blk, blk)], sem)
            for j in range(b_per_w // rep)
        ]
        for h in handles:
            h.wait()

    return k


def kernel(x, emb_table):
    bs, _, seq_len = x.shape
    num_emb, emb_dim = emb_table.shape
    tile_words = seq_len * emb_dim
    out_flat = _sc_broadcast(num_emb * emb_dim, tile_words, bs)(
        emb_table.reshape(-1)
    )
    return out_flat.reshape(bs, tile_words // seq_len, seq_len)


# speedup vs baseline: 2.3295x; 2.3295x over previous
import functools

import jax
import jax.numpy as jnp
from jax.experimental import pallas as pl


def _body(tile_ref, out_ref):
    out_ref[...] = jnp.broadcast_to(tile_ref[...][None], out_ref.shape)


@functools.lru_cache(maxsize=None)
def _bcast(bs, odim, seq_len, blk):
    grid = bs // blk
    return pl.pallas_call(
        _body,
        grid=(grid,),
        in_specs=[pl.BlockSpec((odim, seq_len), lambda i: (0, 0))],
        out_specs=pl.BlockSpec((blk, odim, seq_len), lambda i: (i, 0, 0)),
        out_shape=jax.ShapeDtypeStruct((bs, odim, seq_len), jnp.float32),
    )


def kernel(x, emb_table):
    bs, _, seq_len = x.shape
    emb_dim = emb_table.shape[1]
    tile = emb_table[:seq_len].reshape(emb_dim, seq_len)
    return _bcast(bs, emb_dim, seq_len, 64)(tile)
